# baseline (speedup 1.0000x reference)
import functools

import jax
import jax.numpy as jnp
from jax import lax
from jax.experimental import pallas as pl
from jax.experimental.pallas import tpu as pltpu
from jax.experimental.pallas import tpu_sc as plsc

NUM_CORES = 2
NUM_SUBCORES = 16
LANES = 16
NW = NUM_CORES * NUM_SUBCORES
RING = 8

_MESH = dict(core_axis_name="c", subcore_axis_name="s",
             num_cores=NUM_CORES, num_subcores=NUM_SUBCORES)


def _fm_kernel(d_latent, b_per_w, ui_hbm, ii_hbm, eut_hbm, eit_hbm, b1_hbm,
               h_hbm, out_hbm, ui_v, ii_v, h_v, ub, ib, bb, o_v, *sems):
    wid = lax.axis_index("s") * NUM_CORES + lax.axis_index("c")
    base = wid * b_per_w

    pltpu.sync_copy(ui_hbm.at[pl.ds(base, b_per_w)],
                    ui_v.at[pl.ds(0, b_per_w)])
    pltpu.sync_copy(ii_hbm.at[pl.ds(base, b_per_w)],
                    ii_v.at[pl.ds(0, b_per_w)])
    pltpu.sync_copy(h_hbm, h_v)

    h0 = h_v[pl.ds(0, LANES)]
    h1 = h_v[pl.ds(LANES, LANES)]
    iota = lax.iota(jnp.int32, LANES)
    lane0 = iota == 0

    def idx_vecs(k16):
        fk = jnp.full((LANES,), k16, jnp.int32) + iota
        return plsc.load_gather(ui_v, [fk]), plsc.load_gather(ii_v, [fk])

    def fire(iu, iv, j, slot):
        cu = lax.mul(lax.div(iu[j], 128), 128)
        ci = lax.mul(lax.div(iv[j], 128), 128)
        cb = lax.mul(lax.div(iv[j], LANES), LANES)
        pltpu.async_copy(eut_hbm.at[:, pl.ds(cu, 128)], ub.at[slot],
                         sems[slot])
        pltpu.async_copy(eit_hbm.at[:, pl.ds(ci, 128)], ib.at[slot],
                         sems[slot])
        pltpu.async_copy(b1_hbm.at[pl.ds(cb, LANES)], bb.at[slot],
                         sems[slot])

    iu0, iv0 = idx_vecs(0)
    for j in range(RING):
        fire(iu0, iv0, j, j)

    def step(q, carry):
        iu, iv = idx_vecs(q * RING)
        inext, ivnext = idx_vecs(q * RING + RING)
        for j in range(RING):
            k = q * RING + j
            pltpu.make_async_copy(eut_hbm.at[:, pl.ds(0, 128)], ub.at[j],
                                  sems[j]).wait()
            pltpu.make_async_copy(eit_hbm.at[:, pl.ds(0, 128)], ib.at[j],
                                  sems[j]).wait()
            pltpu.make_async_copy(b1_hbm.at[pl.ds(0, LANES)], bb.at[j],
                                  sems[j]).wait()
            fj = jnp.full((LANES,), j, jnp.int32)
            flu = jnp.full((LANES,), jnp.bitwise_and(iu[j], 127), jnp.int32)
            fli = jnp.full((LANES,), jnp.bitwise_and(iv[j], 127), jnp.int32)
            flb = jnp.full((LANES,), jnp.bitwise_and(iv[j], LANES - 1),
                           jnp.int32)
            u0 = plsc.load_gather(ub, [fj, iota, flu])
            u1 = plsc.load_gather(ub, [fj, iota + LANES, flu])
            v0 = plsc.load_gather(ib, [fj, iota, fli])
            v1 = plsc.load_gather(ib, [fj, iota + LANES, fli])
            t = (jnp.maximum(u0 * v0, 0.0) * h0
                 + jnp.maximum(u1 * v1, 0.0) * h1)
            s = (lax.reduce_sum_p.bind(t, axes=(0,))
                 + plsc.load_gather(bb, [fj, flb])[0])
            plsc.store_scatter(o_v, [jnp.full((LANES,), k, jnp.int32)],
                               jnp.full((LANES,), s, jnp.float32),
                               mask=lane0)

            @pl.when(k + RING < b_per_w)
            def _():
                fire(inext, ivnext, j, j)
        return carry

    lax.fori_loop(0, b_per_w // RING, step, 0)
    pltpu.sync_copy(o_v, out_hbm.at[pl.ds(base, b_per_w)])


def kernel(user_indices, item_indices, embedding_user, embedding_item,
           bias_item, h):
    batch = user_indices.shape[0]
    d_latent = embedding_user.shape[1]
    num_items = bias_item.shape[0]
    assert batch % (NW * RING) == 0 and num_items % LANES == 0
    assert d_latent == 2 * LANES
    b_per_w = batch // NW

    b1 = bias_item.reshape(num_items)
    h1d = h.reshape(d_latent)
    mesh = plsc.VectorSubcoreMesh(**_MESH)

    out = pl.kernel(
        functools.partial(_fm_kernel, d_latent, b_per_w),
        out_type=jax.ShapeDtypeStruct((batch,), jnp.float32),
        mesh=mesh,
        compiler_params=pltpu.CompilerParams(needs_layout_passes=False,
                                             use_tc_tiling_on_sc=True,
                                             disable_bounds_checks=True,
                                             disable_semaphore_checks=True),
        scratch_types=[
            pltpu.VMEM((b_per_w + 2 * LANES,), jnp.int32),
            pltpu.VMEM((b_per_w + 2 * LANES,), jnp.int32),
            pltpu.VMEM((d_latent,), jnp.float32),
            pltpu.VMEM((RING, d_latent, 128), jnp.float32),
            pltpu.VMEM((RING, d_latent, 128), jnp.float32),
            pltpu.VMEM((RING, LANES), jnp.float32),
            pltpu.VMEM((b_per_w,), jnp.float32),
        ] + [pltpu.SemaphoreType.DMA] * RING,
    )(user_indices, item_indices, embedding_user.T, embedding_item.T,
      b1, h1d)
    return out.reshape(batch, 1)
